# Initial kernel scaffold; baseline (speedup 1.0000x reference)
import jax
import jax.numpy as jnp
from jax.experimental import pallas as pl


def kernel(xyz, points, idx, conv_w0, conv_b0, bn_g0, bn_b0, conv_w1, conv_b1, bn_g1, bn_b1):
    raise NotImplementedError("write your pallas kernel here")


# speedup vs baseline: 7.7987x; 7.7987x over previous
import functools

import jax
import jax.numpy as jnp
from jax import lax
from jax.experimental import pallas as pl
from jax.experimental.pallas import tpu as pltpu
from jax.experimental.pallas import tpu_sc as plsc

B, N, S, NS, D = 2, 8192, 2048, 32, 16
C_IN = 3 + D
C1, C2 = 32, 64
EPS = 1e-5
BLK = 256
RBLK = 4096
QBLK = RBLK // NS
NBLK = (B * S * NS) // RBLK
BIG = 3.0e38
HIGH = lax.Precision.HIGHEST


def _k1_body(xyz_ref, pts_ref, w0t_ref, b0_ref, f1_ref):
    x = xyz_ref[0]
    p = pts_ref[0]
    w = w0t_ref[...]
    f = (jnp.dot(x, w[0:3, :], precision=HIGH, preferred_element_type=jnp.float32)
         + jnp.dot(p, w[3:, :], precision=HIGH, preferred_element_type=jnp.float32)
         + b0_ref[...])
    f1_ref[0] = f


def _run_k1(xyz, points, w0t, b0):
    return pl.pallas_call(
        _k1_body,
        grid=(B,),
        in_specs=[
            pl.BlockSpec((1, N, 3), lambda b: (b, 0, 0)),
            pl.BlockSpec((1, N, D), lambda b: (b, 0, 0)),
            pl.BlockSpec((C_IN, C1), lambda b: (0, 0)),
            pl.BlockSpec((1, C1), lambda b: (0, 0)),
        ],
        out_specs=pl.BlockSpec((1, N, C1), lambda b: (b, 0, 0)),
        out_shape=jax.ShapeDtypeStruct((B, N, C1), jnp.float32),
    )(xyz, points, w0t, b0)


def _k2_body(xyz_ref, xyzt_ref, idx_ref, nxyz_ref, knn_ref):
    b = pl.program_id(0)
    xyz = xyz_ref[0]
    idxb = idx_ref[0, 0, :]
    iota = lax.broadcasted_iota(jnp.int32, (BLK, N), 1)
    onehot = (iota == idxb[:, None]).astype(jnp.float32)
    q = jnp.dot(onehot, xyz, precision=HIGH,
                preferred_element_type=jnp.float32)
    nxyz_ref[0] = q

    d0 = q[:, 0:1] - xyzt_ref[0, 0:1, :]
    d1 = q[:, 1:2] - xyzt_ref[0, 1:2, :]
    d2 = q[:, 2:3] - xyzt_ref[0, 2:3, :]
    d = (d0 * d0 + d1 * d1) + d2 * d2

    cols = []
    for _ in range(NS):
        gmin = jnp.min(d, axis=1, keepdims=True)
        cand = jnp.where(d == gmin, iota, N)
        nstar = jnp.min(cand, axis=1, keepdims=True)
        cols.append(nstar)
        d = jnp.where(iota == nstar, BIG, d)
    knn = jnp.concatenate(cols, axis=1) + b * N
    knn_ref[0] = knn


def _run_k2(xyz, xyzt, idx3):
    return pl.pallas_call(
        _k2_body,
        grid=(B, S // BLK),
        in_specs=[
            pl.BlockSpec((1, N, 3), lambda b, j: (b, 0, 0)),
            pl.BlockSpec((1, 3, N), lambda b, j: (b, 0, 0)),
            pl.BlockSpec((1, 1, BLK), lambda b, j: (b * (S // BLK) + j, 0, 0)),
        ],
        out_specs=[
            pl.BlockSpec((1, BLK, 3), lambda b, j: (b, j, 0)),
            pl.BlockSpec((1, BLK, NS), lambda b, j: (b, j, 0)),
        ],
        out_shape=[
            jax.ShapeDtypeStruct((B, S, 3), jnp.float32),
            jax.ShapeDtypeStruct((B, S, NS), jnp.int32),
        ],
    )(xyz, xyzt, idx3)


_IDX_TOTAL = B * S * NS
_GCH = 128


def _make_sc_gather():
    info = plsc.get_sparse_core_info()
    nw = info.num_cores * info.num_subcores
    per_w = _IDX_TOTAL // nw
    nch = per_w // _GCH
    mesh = plsc.VectorSubcoreMesh(core_axis_name="c", subcore_axis_name="s")

    @functools.partial(
        pl.kernel,
        mesh=mesh,
        compiler_params=pltpu.CompilerParams(use_tc_tiling_on_sc=False),
        out_type=jax.ShapeDtypeStruct((_IDX_TOTAL, C1), jnp.float32),
        scratch_types=[
            pltpu.VMEM((per_w,), jnp.int32),
            pltpu.VMEM((_GCH, C1), jnp.float32),
            pltpu.SemaphoreType.DMA,
        ],
    )
    def gather_k(table_hbm, idx_hbm, out_hbm, idx_v, buf0, sem0):
        wid = lax.axis_index("s") * info.num_cores + lax.axis_index("c")
        base = pl.multiple_of(wid * per_w, _GCH)
        pltpu.sync_copy(idx_hbm.at[pl.ds(base, per_w)], idx_v)

        def body(j, _):
            off = pl.multiple_of(j * _GCH, _GCH)
            pltpu.async_copy(
                table_hbm.at[idx_v.at[pl.ds(off, _GCH)]], buf0, sem0).wait()
            dst = pl.multiple_of(base + j * _GCH, _GCH)
            pltpu.sync_copy(buf0, out_hbm.at[pl.ds(dst, _GCH)])
            return 0

        lax.fori_loop(0, nch, body, 0)

    return gather_k


def _k4a_body(g_ref, nx_ref, w0t_ref, s1_ref, q1_ref):
    q = jnp.dot(nx_ref[...], w0t_ref[0:3, :], precision=HIGH,
                preferred_element_type=jnp.float32)
    z1 = g_ref[...].reshape(QBLK, NS, C1) - q[:, None, :]
    s = jnp.sum(z1, axis=(0, 1)).reshape(1, C1)
    sq = jnp.sum(z1 * z1, axis=(0, 1)).reshape(1, C1)

    @pl.when(pl.program_id(0) == 0)
    def _():
        s1_ref[...] = jnp.zeros_like(s1_ref)
        q1_ref[...] = jnp.zeros_like(q1_ref)

    s1_ref[...] += s
    q1_ref[...] += sq


def _run_k4a(gflat, nxflat, w0t):
    return pl.pallas_call(
        _k4a_body,
        grid=(NBLK,),
        in_specs=[
            pl.BlockSpec((RBLK, C1), lambda i: (i, 0)),
            pl.BlockSpec((QBLK, 3), lambda i: (i, 0)),
            pl.BlockSpec((C_IN, C1), lambda i: (0, 0)),
        ],
        out_specs=[
            pl.BlockSpec((1, C1), lambda i: (0, 0)),
            pl.BlockSpec((1, C1), lambda i: (0, 0)),
        ],
        out_shape=[
            jax.ShapeDtypeStruct((1, C1), jnp.float32),
            jax.ShapeDtypeStruct((1, C1), jnp.float32),
        ],
    )(gflat, nxflat, w0t)


def _k4b_body(g_ref, nx_ref, w0t_ref, s1_ref, q1_ref, g0_ref, be0_ref,
              w1t_ref, b1_ref, z2_ref, s2_ref, q2_ref):
    m = jnp.float32(B * S * NS)
    m1 = s1_ref[...] / m
    v1 = q1_ref[...] / m - m1 * m1
    scale = lax.rsqrt(v1 + EPS) * g0_ref[...]
    shift = be0_ref[...] - m1 * scale

    q = jnp.dot(nx_ref[...], w0t_ref[0:3, :], precision=HIGH,
                preferred_element_type=jnp.float32)
    z1 = g_ref[...].reshape(QBLK, NS, C1) - q[:, None, :]
    a1 = jnp.maximum(z1 * scale[None] + shift[None], 0.0)
    z2 = (jnp.dot(a1.reshape(RBLK, C1), w1t_ref[...], precision=HIGH,
                  preferred_element_type=jnp.float32) + b1_ref[...])
    z2_ref[...] = z2
    s = jnp.sum(z2, axis=0).reshape(1, C2)
    sq = jnp.sum(z2 * z2, axis=0).reshape(1, C2)

    @pl.when(pl.program_id(0) == 0)
    def _():
        s2_ref[...] = jnp.zeros_like(s2_ref)
        q2_ref[...] = jnp.zeros_like(q2_ref)

    s2_ref[...] += s
    q2_ref[...] += sq


def _run_k4b(gflat, nxflat, w0t, s1, q1, g0, be0, w1t, b1):
    return pl.pallas_call(
        _k4b_body,
        grid=(NBLK,),
        in_specs=[
            pl.BlockSpec((RBLK, C1), lambda i: (i, 0)),
            pl.BlockSpec((QBLK, 3), lambda i: (i, 0)),
            pl.BlockSpec((C_IN, C1), lambda i: (0, 0)),
            pl.BlockSpec((1, C1), lambda i: (0, 0)),
            pl.BlockSpec((1, C1), lambda i: (0, 0)),
            pl.BlockSpec((1, C1), lambda i: (0, 0)),
            pl.BlockSpec((1, C1), lambda i: (0, 0)),
            pl.BlockSpec((C1, C2), lambda i: (0, 0)),
            pl.BlockSpec((1, C2), lambda i: (0, 0)),
        ],
        out_specs=[
            pl.BlockSpec((RBLK, C2), lambda i: (i, 0)),
            pl.BlockSpec((1, C2), lambda i: (0, 0)),
            pl.BlockSpec((1, C2), lambda i: (0, 0)),
        ],
        out_shape=[
            jax.ShapeDtypeStruct((B * S * NS, C2), jnp.float32),
            jax.ShapeDtypeStruct((1, C2), jnp.float32),
            jax.ShapeDtypeStruct((1, C2), jnp.float32),
        ],
    )(gflat, nxflat, w0t, s1, q1, g0, be0, w1t, b1)


def _k4c_body(z2_ref, s2_ref, q2_ref, g1_ref, be1_ref, out_ref):
    m = jnp.float32(B * S * NS)
    m2 = s2_ref[...] / m
    v2 = q2_ref[...] / m - m2 * m2
    scale = lax.rsqrt(v2 + EPS) * g1_ref[...]
    shift = be1_ref[...] - m2 * scale
    a2 = jnp.maximum(z2_ref[...] * scale + shift, 0.0)
    out_ref[...] = jnp.max(a2.reshape(QBLK, NS, C2), axis=1)


def _run_k4c(z2, s2, q2, g1, be1):
    return pl.pallas_call(
        _k4c_body,
        grid=(NBLK,),
        in_specs=[
            pl.BlockSpec((RBLK, C2), lambda i: (i, 0)),
            pl.BlockSpec((1, C2), lambda i: (0, 0)),
            pl.BlockSpec((1, C2), lambda i: (0, 0)),
            pl.BlockSpec((1, C2), lambda i: (0, 0)),
            pl.BlockSpec((1, C2), lambda i: (0, 0)),
        ],
        out_specs=pl.BlockSpec((QBLK, C2), lambda i: (i, 0)),
        out_shape=jax.ShapeDtypeStruct((B * S, C2), jnp.float32),
    )(z2, s2, q2, g1, be1)


def kernel(xyz, points, idx, conv_w0, conv_b0, bn_g0, bn_b0,
           conv_w1, conv_b1, bn_g1, bn_b1):
    w0t = conv_w0.T
    w1t = conv_w1.T
    b0 = conv_b0.reshape(1, C1)
    b1 = conv_b1.reshape(1, C2)
    g0 = bn_g0.reshape(1, C1)
    be0 = bn_b0.reshape(1, C1)
    g1 = bn_g1.reshape(1, C2)
    be1 = bn_b1.reshape(1, C2)

    f1 = _run_k1(xyz, points, w0t, b0)
    xyzt = jnp.transpose(xyz, (0, 2, 1))
    idx3 = idx.reshape(B * (S // BLK), 1, BLK)
    new_xyz, knn = _run_k2(xyz, xyzt, idx3)

    gather_k = _make_sc_gather()
    gflat = gather_k(f1.reshape(B * N, C1), knn.reshape(_IDX_TOTAL))

    nxflat = new_xyz.reshape(B * S, 3)
    s1, q1 = _run_k4a(gflat, nxflat, w0t)
    z2, s2, q2 = _run_k4b(gflat, nxflat, w0t, s1, q1, g0, be0, w1t, b1)
    out = _run_k4c(z2, s2, q2, g1, be1)
    return (new_xyz, out.reshape(B, S, C2))
